# Initial kernel scaffold; baseline (speedup 1.0000x reference)
import functools

import jax
import jax.numpy as jnp
from jax import lax
from jax.experimental import pallas as pl
from jax.experimental.pallas import tpu as pltpu
from jax.experimental.pallas import tpu_sc as plsc

NUM_CORES = 2
NUM_SUBCORES = 16
NUM_WORKERS = NUM_CORES * NUM_SUBCORES


def _pick_batch(edges_per_worker):
    for cand in range(128, 0, -1):
        if edges_per_worker % cand == 0:
            return cand
    return 1


def _sc_degree(src2d, dst2d, zeros16, ones_rows, n):
    nbt, b = src2d.shape
    nbw = nbt // NUM_WORKERS
    rows_per_sub = n // NUM_SUBCORES
    mesh = plsc.VectorSubcoreMesh(core_axis_name="c", subcore_axis_name="s")

    @functools.partial(
        pl.kernel,
        out_type=jax.ShapeDtypeStruct((2, NUM_CORES, n, 16), jnp.float32),
        mesh=mesh,
        scratch_types=[
            pltpu.VMEM((nbw, b), jnp.int32),
            pltpu.VMEM((nbw, b), jnp.int32),
            pltpu.VMEM((b, 16), jnp.float32),
            pltpu.VMEM_SHARED((n, 16), jnp.float32),
            pltpu.VMEM_SHARED((n, 16), jnp.float32),
            pltpu.SemaphoreType.DMA,
        ],
    )
    def deg_kernel(src_hbm, dst_hbm, z_hbm, ones_hbm, out_hbm,
                   sidx, didx, ones_v, dsrc_sh, ddst_sh, sem):
        c = lax.axis_index("c")
        s = lax.axis_index("s")
        wid = c * NUM_SUBCORES + s
        r0 = s * rows_per_sub
        pltpu.sync_copy(src_hbm.at[pl.ds(wid * nbw, nbw)], sidx)
        pltpu.sync_copy(dst_hbm.at[pl.ds(wid * nbw, nbw)], didx)
        pltpu.sync_copy(ones_hbm, ones_v)
        pltpu.sync_copy(z_hbm.at[pl.ds(r0, rows_per_sub)],
                        dsrc_sh.at[pl.ds(r0, rows_per_sub)])
        pltpu.sync_copy(z_hbm.at[pl.ds(r0, rows_per_sub)],
                        ddst_sh.at[pl.ds(r0, rows_per_sub)])
        plsc.subcore_barrier()

        @pl.loop(0, nbw)
        def _(i):
            pltpu.sync_copy(ones_v, dsrc_sh.at[sidx.at[i]], add=True)
            pltpu.sync_copy(ones_v, ddst_sh.at[didx.at[i]], add=True)

        plsc.subcore_barrier()
        pltpu.sync_copy(dsrc_sh.at[pl.ds(r0, rows_per_sub)],
                        out_hbm.at[0, c, pl.ds(r0, rows_per_sub)])
        pltpu.sync_copy(ddst_sh.at[pl.ds(r0, rows_per_sub)],
                        out_hbm.at[1, c, pl.ds(r0, rows_per_sub)])

    return deg_kernel(src2d, dst2d, zeros16, ones_rows)


def _sc_prop(u, src2d, dst2d, zeros, n, cp):
    nbt, b = src2d.shape
    nbw = nbt // NUM_WORKERS
    rows_per_sub = n // NUM_SUBCORES
    mesh = plsc.VectorSubcoreMesh(core_axis_name="c", subcore_axis_name="s")

    @functools.partial(
        pl.kernel,
        out_type=jax.ShapeDtypeStruct((NUM_CORES, n, cp), jnp.float32),
        mesh=mesh,
        scratch_types=[
            pltpu.VMEM((nbw, b), jnp.int32),
            pltpu.VMEM((nbw, b), jnp.int32),
            pltpu.VMEM((b, cp), jnp.float32),
            pltpu.VMEM((b, cp), jnp.float32),
            pltpu.VMEM_SHARED((n, cp), jnp.float32),
            pltpu.SemaphoreType.DMA,
            pltpu.SemaphoreType.DMA,
        ],
    )
    def hop_kernel(u_hbm, src_hbm, dst_hbm, z_hbm, out_hbm,
                   sidx, didx, rows_a, rows_b, acc_sh, sem_a, sem_b):
        c = lax.axis_index("c")
        s = lax.axis_index("s")
        wid = c * NUM_SUBCORES + s
        r0 = s * rows_per_sub
        pltpu.sync_copy(src_hbm.at[pl.ds(wid * nbw, nbw)], sidx)
        pltpu.sync_copy(dst_hbm.at[pl.ds(wid * nbw, nbw)], didx)
        pltpu.sync_copy(z_hbm.at[pl.ds(r0, rows_per_sub)],
                        acc_sh.at[pl.ds(r0, rows_per_sub)])
        plsc.subcore_barrier()

        pltpu.async_copy(u_hbm.at[sidx.at[0]], rows_a, sem_a)
        pltpu.async_copy(u_hbm.at[sidx.at[1]], rows_b, sem_b)

        @pl.loop(0, nbw, step=2)
        def _(g):
            pltpu.make_async_copy(u_hbm.at[sidx.at[g]], rows_a, sem_a).wait()
            pltpu.sync_copy(rows_a, acc_sh.at[didx.at[g]], add=True)

            @pl.when(g + 2 < nbw)
            def _():
                pltpu.async_copy(u_hbm.at[sidx.at[g + 2]], rows_a, sem_a)

            pltpu.make_async_copy(u_hbm.at[sidx.at[g + 1]], rows_b, sem_b).wait()
            pltpu.sync_copy(rows_b, acc_sh.at[didx.at[g + 1]], add=True)

            @pl.when(g + 3 < nbw)
            def _():
                pltpu.async_copy(u_hbm.at[sidx.at[g + 3]], rows_b, sem_b)

        plsc.subcore_barrier()
        pltpu.sync_copy(acc_sh.at[pl.ds(r0, rows_per_sub)],
                        out_hbm.at[c, pl.ds(r0, rows_per_sub)])

    return hop_kernel(u, src2d, dst2d, zeros)


def _tc_project(feature, w_padded, n, cp):
    def body(f_ref, w_ref, o_ref):
        o_ref[...] = jnp.dot(f_ref[...], w_ref[...],
                             preferred_element_type=jnp.float32)

    return pl.pallas_call(
        body, out_shape=jax.ShapeDtypeStruct((n, cp), jnp.float32),
    )(feature, w_padded)


def _tc_prep(y0, d_out0, d_out1, d_in0, d_in1, n, cp):
    def body(y_ref, do0, do1, di0, di1, u_ref, rin_ref, rout_ref):
        deg_out = do0[...] + do1[...]
        deg_in = di0[...] + di1[...]
        rout = lax.rsqrt(jnp.maximum(deg_out, 1.0))
        rin = lax.rsqrt(jnp.maximum(deg_in, 1.0))
        u_ref[...] = y_ref[...] * rout
        rin_ref[...] = rin
        rout_ref[...] = rout

    return pl.pallas_call(
        body,
        out_shape=[
            jax.ShapeDtypeStruct((n, cp), jnp.float32),
            jax.ShapeDtypeStruct((n, 1), jnp.float32),
            jax.ShapeDtypeStruct((n, 1), jnp.float32),
        ],
    )(y0, d_out0, d_out1, d_in0, d_in1)


def _tc_step(p, rin, rout, acc, n, cp):
    def body(p_ref, rin_ref, rout_ref, acc_ref, u_ref, accout_ref):
        x = (p_ref[0] + p_ref[1]) * rin_ref[...]
        u_ref[...] = x * rout_ref[...]
        accout_ref[...] = acc_ref[...] + x

    return pl.pallas_call(
        body,
        out_shape=[
            jax.ShapeDtypeStruct((n, cp), jnp.float32),
            jax.ShapeDtypeStruct((n, cp), jnp.float32),
        ],
    )(p, rin, rout, acc)


def _tc_final(p, rin, acc, b2d, n, c_out):
    def body(p_ref, rin_ref, acc_ref, b_ref, o_ref):
        x = (p_ref[0] + p_ref[1]) * rin_ref[...]
        res = (acc_ref[...] + x) * 0.25
        o_ref[...] = res[:, :c_out] + b_ref[...]

    return pl.pallas_call(
        body, out_shape=jax.ShapeDtypeStruct((n, c_out), jnp.float32),
    )(p, rin, acc, b2d)


def kernel(feature, edge_index, W, b):
    n, d = feature.shape
    c_out = W.shape[1]
    e = edge_index.shape[1]
    cp = ((c_out + 15) // 16) * 16

    epw = e // NUM_WORKERS
    batch = _pick_batch(epw)
    nbt = e // batch

    src2d = edge_index[0].reshape(nbt, batch)
    dst2d = edge_index[1].reshape(nbt, batch)
    w_padded = jnp.pad(W, ((0, 0), (0, cp - c_out)))
    zeros16 = jnp.zeros((n, 16), jnp.float32)
    ones_rows = jnp.ones((batch, 16), jnp.float32)
    zeros_cp = jnp.zeros((n, cp), jnp.float32)
    b2d = b.reshape(1, c_out)

    y0 = _tc_project(feature, w_padded, n, cp)
    degs = _sc_degree(src2d, dst2d, zeros16, ones_rows, n)
    d_out0 = degs[0, 0, :, 0:1]
    d_out1 = degs[0, 1, :, 0:1]
    d_in0 = degs[1, 0, :, 0:1]
    d_in1 = degs[1, 1, :, 0:1]

    u, rin, rout = _tc_prep(y0, d_out0, d_out1, d_in0, d_in1, n, cp)
    acc = y0
    for hop in range(3):
        p = _sc_prop(u, src2d, dst2d, zeros_cp, n, cp)
        if hop < 2:
            u, acc = _tc_step(p, rin, rout, acc, n, cp)
        else:
            out = _tc_final(p, rin, acc, b2d, n, c_out)
    return out


# speedup vs baseline: 22.6828x; 22.6828x over previous
import functools

import jax
import jax.numpy as jnp
from jax import lax
from jax.experimental import pallas as pl
from jax.experimental.pallas import tpu as pltpu
from jax.experimental.pallas import tpu_sc as plsc

NUM_CORES = 2
NUM_SUBCORES = 16
NUM_WORKERS = NUM_CORES * NUM_SUBCORES

_SC_PARAMS = pltpu.CompilerParams(use_tc_tiling_on_sc=False)


def _pick_batch(edges_per_worker):
    for cand in range(128, 0, -1):
        if edges_per_worker % cand == 0:
            return cand
    return 1


def _row_chunks(n):
    for nz in range(NUM_SUBCORES, 0, -1):
        if n % nz == 0 and (n // nz) % 8 == 0:
            return nz, n // nz
    return 1, n


def _sc_degree(src2d, dst2d, zeros16, ones_rows, n):
    nbt, b = src2d.shape
    nbw = nbt // NUM_WORKERS
    nz, zch = _row_chunks(n)
    mesh = plsc.VectorSubcoreMesh(core_axis_name="c", subcore_axis_name="s")

    @functools.partial(
        pl.kernel,
        out_type=jax.ShapeDtypeStruct((2, NUM_CORES, n, 16), jnp.float32),
        mesh=mesh,
        scratch_types=[
            pltpu.VMEM((nbw, b), jnp.int32),
            pltpu.VMEM((nbw, b), jnp.int32),
            pltpu.VMEM((b, 16), jnp.float32),
            pltpu.VMEM_SHARED((n, 16), jnp.float32),
            pltpu.VMEM_SHARED((n, 16), jnp.float32),
            pltpu.SemaphoreType.DMA,
        ],
        compiler_params=_SC_PARAMS,
    )
    def deg_kernel(src_hbm, dst_hbm, z_hbm, ones_hbm, out_hbm,
                   sidx, didx, ones_v, dsrc_sh, ddst_sh, sem):
        c = lax.axis_index("c")
        s = lax.axis_index("s")
        wid = c * NUM_SUBCORES + s
        r0 = s * zch
        pltpu.sync_copy(src_hbm.at[pl.ds(wid * nbw, nbw)], sidx)
        pltpu.sync_copy(dst_hbm.at[pl.ds(wid * nbw, nbw)], didx)
        pltpu.sync_copy(ones_hbm, ones_v)

        @pl.when(s < nz)
        def _():
            pltpu.sync_copy(z_hbm.at[pl.ds(r0, zch)],
                            dsrc_sh.at[pl.ds(r0, zch)])
            pltpu.sync_copy(z_hbm.at[pl.ds(r0, zch)],
                            ddst_sh.at[pl.ds(r0, zch)])

        plsc.subcore_barrier()

        @pl.loop(0, nbw)
        def _(i):
            pltpu.sync_copy(ones_v, dsrc_sh.at[sidx.at[i]], add=True)
            pltpu.sync_copy(ones_v, ddst_sh.at[didx.at[i]], add=True)

        plsc.subcore_barrier()

        @pl.when(s < nz)
        def _():
            pltpu.sync_copy(dsrc_sh.at[pl.ds(r0, zch)],
                            out_hbm.at[0, c, pl.ds(r0, zch)])
            pltpu.sync_copy(ddst_sh.at[pl.ds(r0, zch)],
                            out_hbm.at[1, c, pl.ds(r0, zch)])

    return deg_kernel(src2d, dst2d, zeros16, ones_rows)


def _sc_prop(u, src2d, dst2d, zeros, n, cp):
    nbt, b = src2d.shape
    nbw = nbt // NUM_WORKERS
    nz, zch = _row_chunks(n)
    mesh = plsc.VectorSubcoreMesh(core_axis_name="c", subcore_axis_name="s")

    @functools.partial(
        pl.kernel,
        out_type=jax.ShapeDtypeStruct((NUM_CORES, n, cp), jnp.float32),
        mesh=mesh,
        scratch_types=[
            pltpu.VMEM((nbw, b), jnp.int32),
            pltpu.VMEM((nbw, b), jnp.int32),
            pltpu.VMEM((b, cp), jnp.float32),
            pltpu.VMEM((b, cp), jnp.float32),
            pltpu.VMEM_SHARED((n, cp), jnp.float32),
            pltpu.SemaphoreType.DMA,
            pltpu.SemaphoreType.DMA,
        ],
        compiler_params=_SC_PARAMS,
    )
    def hop_kernel(u_hbm, src_hbm, dst_hbm, z_hbm, out_hbm,
                   sidx, didx, rows_a, rows_b, acc_sh, sem_a, sem_b):
        c = lax.axis_index("c")
        s = lax.axis_index("s")
        wid = c * NUM_SUBCORES + s
        r0 = s * zch
        pltpu.sync_copy(src_hbm.at[pl.ds(wid * nbw, nbw)], sidx)
        pltpu.sync_copy(dst_hbm.at[pl.ds(wid * nbw, nbw)], didx)

        @pl.when(s < nz)
        def _():
            pltpu.sync_copy(z_hbm.at[pl.ds(r0, zch)],
                            acc_sh.at[pl.ds(r0, zch)])

        plsc.subcore_barrier()

        pltpu.async_copy(u_hbm.at[sidx.at[0]], rows_a, sem_a)
        pltpu.async_copy(u_hbm.at[sidx.at[1]], rows_b, sem_b)

        @pl.loop(0, nbw, step=2)
        def _(g):
            pltpu.make_async_copy(u_hbm.at[sidx.at[g]], rows_a, sem_a).wait()
            pltpu.sync_copy(rows_a, acc_sh.at[didx.at[g]], add=True)

            @pl.when(g + 2 < nbw)
            def _():
                pltpu.async_copy(u_hbm.at[sidx.at[g + 2]], rows_a, sem_a)

            pltpu.make_async_copy(u_hbm.at[sidx.at[g + 1]], rows_b, sem_b).wait()
            pltpu.sync_copy(rows_b, acc_sh.at[didx.at[g + 1]], add=True)

            @pl.when(g + 3 < nbw)
            def _():
                pltpu.async_copy(u_hbm.at[sidx.at[g + 3]], rows_b, sem_b)

        plsc.subcore_barrier()

        @pl.when(s < nz)
        def _():
            pltpu.sync_copy(acc_sh.at[pl.ds(r0, zch)],
                            out_hbm.at[c, pl.ds(r0, zch)])

    return hop_kernel(u, src2d, dst2d, zeros)


def _tc_project(feature, w_padded, n, cp):
    def body(f_ref, w_ref, o_ref):
        o_ref[...] = jnp.dot(f_ref[...], w_ref[...],
                             preferred_element_type=jnp.float32)

    return pl.pallas_call(
        body, out_shape=jax.ShapeDtypeStruct((n, cp), jnp.float32),
    )(feature, w_padded)


def _tc_prep(y0, d_out0, d_out1, d_in0, d_in1, n, cp):
    def body(y_ref, do0, do1, di0, di1, u_ref, rin_ref, rout_ref):
        deg_out = do0[...] + do1[...]
        deg_in = di0[...] + di1[...]
        rout = lax.rsqrt(jnp.maximum(deg_out, 1.0))
        rin = lax.rsqrt(jnp.maximum(deg_in, 1.0))
        u_ref[...] = y_ref[...] * rout
        rin_ref[...] = rin
        rout_ref[...] = rout

    return pl.pallas_call(
        body,
        out_shape=[
            jax.ShapeDtypeStruct((n, cp), jnp.float32),
            jax.ShapeDtypeStruct((n, 1), jnp.float32),
            jax.ShapeDtypeStruct((n, 1), jnp.float32),
        ],
    )(y0, d_out0, d_out1, d_in0, d_in1)


def _tc_step(p, rin, rout, acc, n, cp):
    def body(p_ref, rin_ref, rout_ref, acc_ref, u_ref, accout_ref):
        x = (p_ref[0] + p_ref[1]) * rin_ref[...]
        u_ref[...] = x * rout_ref[...]
        accout_ref[...] = acc_ref[...] + x

    return pl.pallas_call(
        body,
        out_shape=[
            jax.ShapeDtypeStruct((n, cp), jnp.float32),
            jax.ShapeDtypeStruct((n, cp), jnp.float32),
        ],
    )(p, rin, rout, acc)


def _tc_final(p, rin, acc, b2d, n, c_out):
    def body(p_ref, rin_ref, acc_ref, b_ref, o_ref):
        x = (p_ref[0] + p_ref[1]) * rin_ref[...]
        res = (acc_ref[...] + x) * 0.25
        o_ref[...] = res[:, :c_out] + b_ref[...]

    return pl.pallas_call(
        body, out_shape=jax.ShapeDtypeStruct((n, c_out), jnp.float32),
    )(p, rin, acc, b2d)


def kernel(feature, edge_index, W, b):
    n, d = feature.shape
    c_out = W.shape[1]
    e = edge_index.shape[1]
    cp = ((c_out + 15) // 16) * 16

    epw = e // NUM_WORKERS
    batch = _pick_batch(epw)
    nbt = e // batch

    src2d = edge_index[0].reshape(nbt, batch)
    dst2d = edge_index[1].reshape(nbt, batch)
    w_padded = jnp.pad(W, ((0, 0), (0, cp - c_out)))
    zeros16 = jnp.zeros((n, 16), jnp.float32)
    ones_rows = jnp.ones((batch, 16), jnp.float32)
    zeros_cp = jnp.zeros((n, cp), jnp.float32)
    b2d = b.reshape(1, c_out)

    y0 = _tc_project(feature, w_padded, n, cp)
    degs = _sc_degree(src2d, dst2d, zeros16, ones_rows, n)
    d_out0 = degs[0, 0, :, 0:1]
    d_out1 = degs[0, 1, :, 0:1]
    d_in0 = degs[1, 0, :, 0:1]
    d_in1 = degs[1, 1, :, 0:1]

    u, rin, rout = _tc_prep(y0, d_out0, d_out1, d_in0, d_in1, n, cp)
    acc = y0
    for hop in range(3):
        p = _sc_prop(u, src2d, dst2d, zeros_cp, n, cp)
        if hop < 2:
            u, acc = _tc_step(p, rin, rout, acc, n, cp)
        else:
            out = _tc_final(p, rin, acc, b2d, n, c_out)
    return out


# speedup vs baseline: 26.6268x; 1.1739x over previous
import functools

import jax
import jax.numpy as jnp
from jax import lax
from jax.experimental import pallas as pl
from jax.experimental.pallas import tpu as pltpu
from jax.experimental.pallas import tpu_sc as plsc

NUM_CORES = 2
NUM_SUBCORES = 16
NUM_WORKERS = NUM_CORES * NUM_SUBCORES

_SC_PARAMS = pltpu.CompilerParams(use_tc_tiling_on_sc=False)


def _pick_batch(edges_per_worker):
    for cand in range(128, 0, -1):
        if edges_per_worker % cand == 0:
            return cand
    return 1


def _row_chunks(n):
    for nz in range(NUM_SUBCORES, 0, -1):
        if n % nz == 0 and (n // nz) % 8 == 0:
            return nz, n // nz
    return 1, n


def _sc_degree(src2d, dst2d, zeros16, ones_rows, n):
    nbt, b = src2d.shape
    nbw = nbt // NUM_WORKERS
    nz, zch = _row_chunks(n)
    mesh = plsc.VectorSubcoreMesh(core_axis_name="c", subcore_axis_name="s")

    @functools.partial(
        pl.kernel,
        out_type=jax.ShapeDtypeStruct((2, NUM_CORES, n, 16), jnp.float32),
        mesh=mesh,
        scratch_types=[
            pltpu.VMEM((nbw, b), jnp.int32),
            pltpu.VMEM((nbw, b), jnp.int32),
            pltpu.VMEM((b, 16), jnp.float32),
            pltpu.VMEM_SHARED((n, 16), jnp.float32),
            pltpu.VMEM_SHARED((n, 16), jnp.float32),
            pltpu.SemaphoreType.DMA,
        ],
        compiler_params=_SC_PARAMS,
    )
    def deg_kernel(src_hbm, dst_hbm, z_hbm, ones_hbm, out_hbm,
                   sidx, didx, ones_v, dsrc_sh, ddst_sh, sem):
        c = lax.axis_index("c")
        s = lax.axis_index("s")
        wid = c * NUM_SUBCORES + s
        r0 = s * zch
        pltpu.sync_copy(src_hbm.at[pl.ds(wid * nbw, nbw)], sidx)
        pltpu.sync_copy(dst_hbm.at[pl.ds(wid * nbw, nbw)], didx)
        pltpu.sync_copy(ones_hbm, ones_v)

        @pl.when(s < nz)
        def _():
            pltpu.sync_copy(z_hbm.at[pl.ds(r0, zch)],
                            dsrc_sh.at[pl.ds(r0, zch)])
            pltpu.sync_copy(z_hbm.at[pl.ds(r0, zch)],
                            ddst_sh.at[pl.ds(r0, zch)])

        plsc.subcore_barrier()

        @pl.loop(0, nbw)
        def _(i):
            pltpu.async_copy(ones_v, dsrc_sh.at[sidx.at[i]], sem, add=True)
            pltpu.async_copy(ones_v, ddst_sh.at[didx.at[i]], sem, add=True)

        @pl.loop(0, 2 * nbw)
        def _(i):
            pltpu.make_async_copy(ones_v, dsrc_sh.at[sidx.at[0]], sem).wait()

        plsc.subcore_barrier()

        @pl.when(s < nz)
        def _():
            pltpu.sync_copy(dsrc_sh.at[pl.ds(r0, zch)],
                            out_hbm.at[0, c, pl.ds(r0, zch)])
            pltpu.sync_copy(ddst_sh.at[pl.ds(r0, zch)],
                            out_hbm.at[1, c, pl.ds(r0, zch)])

    return deg_kernel(src2d, dst2d, zeros16, ones_rows)


def _sc_prop(u, src2d, dst2d, zeros, n, cp):
    nbt, b = src2d.shape
    nbw = nbt // NUM_WORKERS
    nz, zch = _row_chunks(n)
    mesh = plsc.VectorSubcoreMesh(core_axis_name="c", subcore_axis_name="s")

    @functools.partial(
        pl.kernel,
        out_type=jax.ShapeDtypeStruct((NUM_CORES, n, cp), jnp.float32),
        mesh=mesh,
        scratch_types=[
            pltpu.VMEM((nbw, b), jnp.int32),
            pltpu.VMEM((nbw, b), jnp.int32),
            pltpu.VMEM((b, cp), jnp.float32),
            pltpu.VMEM((b, cp), jnp.float32),
            pltpu.VMEM((b, cp), jnp.float32),
            pltpu.VMEM((b, cp), jnp.float32),
            pltpu.VMEM_SHARED((n, cp), jnp.float32),
            pltpu.SemaphoreType.DMA,
            pltpu.SemaphoreType.DMA,
            pltpu.SemaphoreType.DMA,
            pltpu.SemaphoreType.DMA,
            pltpu.SemaphoreType.DMA,
            pltpu.SemaphoreType.DMA,
            pltpu.SemaphoreType.DMA,
            pltpu.SemaphoreType.DMA,
        ],
        compiler_params=_SC_PARAMS,
    )
    def hop_kernel(u_hbm, src_hbm, dst_hbm, z_hbm, out_hbm,
                   sidx, didx, r0b, r1b, r2b, r3b, acc_sh,
                   g0, g1, g2, g3, s0, s1, s2, s3):
        rows = (r0b, r1b, r2b, r3b)
        gsem = (g0, g1, g2, g3)
        ssem = (s0, s1, s2, s3)
        c = lax.axis_index("c")
        s = lax.axis_index("s")
        wid = c * NUM_SUBCORES + s
        r0 = s * zch
        pltpu.sync_copy(src_hbm.at[pl.ds(wid * nbw, nbw)], sidx)
        pltpu.sync_copy(dst_hbm.at[pl.ds(wid * nbw, nbw)], didx)

        @pl.when(s < nz)
        def _():
            pltpu.sync_copy(z_hbm.at[pl.ds(r0, zch)],
                            acc_sh.at[pl.ds(r0, zch)])

        plsc.subcore_barrier()

        nv = nbw + 4

        @pl.loop(0, nv, step=4)
        def _(v):
            for j in range(4):
                i = v + j

                @pl.when((i >= 4) & (i < nbw))
                def _():
                    pltpu.make_async_copy(
                        rows[j], acc_sh.at[didx.at[0]], ssem[j]).wait()

                @pl.when(i < nbw)
                def _():
                    pltpu.async_copy(u_hbm.at[sidx.at[i]], rows[j], gsem[j])

                k = i - 2
                jb = (j + 2) % 4

                @pl.when((k >= 0) & (k < nbw))
                def _():
                    pltpu.make_async_copy(
                        u_hbm.at[sidx.at[0]], rows[jb], gsem[jb]).wait()
                    pltpu.async_copy(rows[jb], acc_sh.at[didx.at[k]],
                                     ssem[jb], add=True)

        for j in range(4):
            pltpu.make_async_copy(rows[j], acc_sh.at[didx.at[0]],
                                  ssem[j]).wait()

        plsc.subcore_barrier()

        @pl.when(s < nz)
        def _():
            pltpu.sync_copy(acc_sh.at[pl.ds(r0, zch)],
                            out_hbm.at[c, pl.ds(r0, zch)])

    return hop_kernel(u, src2d, dst2d, zeros)


def _tc_project(feature, w_padded, n, cp):
    def body(f_ref, w_ref, o_ref):
        o_ref[...] = jnp.dot(f_ref[...], w_ref[...],
                             preferred_element_type=jnp.float32)

    return pl.pallas_call(
        body, out_shape=jax.ShapeDtypeStruct((n, cp), jnp.float32),
    )(feature, w_padded)


def _tc_prep(y0, d_out0, d_out1, d_in0, d_in1, n, cp):
    def body(y_ref, do0, do1, di0, di1, u_ref, rin_ref, rout_ref):
        deg_out = do0[...] + do1[...]
        deg_in = di0[...] + di1[...]
        rout = lax.rsqrt(jnp.maximum(deg_out, 1.0))
        rin = lax.rsqrt(jnp.maximum(deg_in, 1.0))
        u_ref[...] = y_ref[...] * rout
        rin_ref[...] = rin
        rout_ref[...] = rout

    return pl.pallas_call(
        body,
        out_shape=[
            jax.ShapeDtypeStruct((n, cp), jnp.float32),
            jax.ShapeDtypeStruct((n, 1), jnp.float32),
            jax.ShapeDtypeStruct((n, 1), jnp.float32),
        ],
    )(y0, d_out0, d_out1, d_in0, d_in1)


def _tc_step(p, rin, rout, acc, n, cp):
    def body(p_ref, rin_ref, rout_ref, acc_ref, u_ref, accout_ref):
        x = (p_ref[0] + p_ref[1]) * rin_ref[...]
        u_ref[...] = x * rout_ref[...]
        accout_ref[...] = acc_ref[...] + x

    return pl.pallas_call(
        body,
        out_shape=[
            jax.ShapeDtypeStruct((n, cp), jnp.float32),
            jax.ShapeDtypeStruct((n, cp), jnp.float32),
        ],
    )(p, rin, rout, acc)


def _tc_final(p, rin, acc, b2d, n, c_out):
    def body(p_ref, rin_ref, acc_ref, b_ref, o_ref):
        x = (p_ref[0] + p_ref[1]) * rin_ref[...]
        res = (acc_ref[...] + x) * 0.25
        o_ref[...] = res[:, :c_out] + b_ref[...]

    return pl.pallas_call(
        body, out_shape=jax.ShapeDtypeStruct((n, c_out), jnp.float32),
    )(p, rin, acc, b2d)


def kernel(feature, edge_index, W, b):
    n, d = feature.shape
    c_out = W.shape[1]
    e = edge_index.shape[1]
    cp = ((c_out + 15) // 16) * 16

    epw = e // NUM_WORKERS
    batch = _pick_batch(epw)
    nbt = e // batch

    src2d = edge_index[0].reshape(nbt, batch)
    dst2d = edge_index[1].reshape(nbt, batch)
    w_padded = jnp.pad(W, ((0, 0), (0, cp - c_out)))
    zeros16 = jnp.zeros((n, 16), jnp.float32)
    ones_rows = jnp.ones((batch, 16), jnp.float32)
    zeros_cp = jnp.zeros((n, cp), jnp.float32)
    b2d = b.reshape(1, c_out)

    y0 = _tc_project(feature, w_padded, n, cp)
    degs = _sc_degree(src2d, dst2d, zeros16, ones_rows, n)
    d_out0 = degs[0, 0, :, 0:1]
    d_out1 = degs[0, 1, :, 0:1]
    d_in0 = degs[1, 0, :, 0:1]
    d_in1 = degs[1, 1, :, 0:1]

    u, rin, rout = _tc_prep(y0, d_out0, d_out1, d_in0, d_in1, n, cp)
    acc = y0
    for hop in range(3):
        p = _sc_prop(u, src2d, dst2d, zeros_cp, n, cp)
        if hop < 2:
            u, acc = _tc_step(p, rin, rout, acc, n, cp)
        else:
            out = _tc_final(p, rin, acc, b2d, n, c_out)
    return out
